# baseline (device time: 41124 ns/iter reference)
import jax
import jax.numpy as jnp
from jax import lax
from jax.experimental import pallas as pl
from jax.experimental.pallas import tpu as pltpu

N_DEV = 16
B = 512
D = 256
CHUNK = B // N_DEV


def kernel(x, Win0, Wout0, Win1, Wout1, Win2, Wout2):
    def body(x_ref, win0_ref, wout0_ref, win1_ref, wout1_ref,
             win2_ref, wout2_ref, out_ref,
             partial_ref, rs_buf, y_ref, xnext_ref,
             send_sems, rs_sems, ag_sems):
        me = lax.axis_index("i")

        barrier = pltpu.get_barrier_semaphore()
        for d in range(N_DEV):
            @pl.when(d != me)
            def _():
                pl.semaphore_signal(
                    barrier, inc=1, device_id=(d,),
                    device_id_type=pl.DeviceIdType.MESH,
                )
        pl.semaphore_wait(barrier, N_DEV - 1)

        def send_desc_rs(d):
            return pltpu.make_async_remote_copy(
                src_ref=partial_ref.at[pl.ds(d * CHUNK, CHUNK), :],
                dst_ref=rs_buf.at[me],
                send_sem=send_sems.at[d],
                recv_sem=rs_sems.at[me],
                device_id=(d,),
                device_id_type=pl.DeviceIdType.MESH,
            )

        def recv_desc_rs(d):
            return pltpu.make_async_remote_copy(
                src_ref=rs_buf.at[d],
                dst_ref=rs_buf.at[d],
                send_sem=send_sems.at[d],
                recv_sem=rs_sems.at[d],
                device_id=(d,),
                device_id_type=pl.DeviceIdType.MESH,
            )

        def reduce_scatter(p32):
            partial_ref[:, :] = p32.astype(jnp.bfloat16)
            for d in range(N_DEV):
                @pl.when(d != me)
                def _():
                    send_desc_rs(d).start()
            for d in range(N_DEV):
                @pl.when(d != me)
                def _():
                    send_desc_rs(d).wait_send()
            acc = partial_ref[pl.ds(me * CHUNK, CHUNK), :].astype(jnp.float32)
            for d in range(N_DEV):
                @pl.when(d != me)
                def _():
                    recv_desc_rs(d).wait_recv()
            for d in range(N_DEV):
                acc = acc + jnp.where(
                    d == me,
                    jnp.zeros((CHUNK, D), jnp.float32),
                    rs_buf[d, :, :].astype(jnp.float32),
                )
            return acc

        def send_desc_ag(d):
            return pltpu.make_async_remote_copy(
                src_ref=y_ref,
                dst_ref=xnext_ref.at[pl.ds(me * CHUNK, CHUNK), :],
                send_sem=send_sems.at[d],
                recv_sem=ag_sems.at[me],
                device_id=(d,),
                device_id_type=pl.DeviceIdType.MESH,
            )

        def recv_desc_ag(d):
            return pltpu.make_async_remote_copy(
                src_ref=y_ref,
                dst_ref=xnext_ref.at[pl.ds(d * CHUNK, CHUNK), :],
                send_sem=send_sems.at[d],
                recv_sem=ag_sems.at[d],
                device_id=(d,),
                device_id_type=pl.DeviceIdType.MESH,
            )

        def all_gather(acc):
            yb = acc.astype(jnp.bfloat16)
            y_ref[:, :] = yb
            for d in range(N_DEV):
                @pl.when(d != me)
                def _():
                    send_desc_ag(d).start()
            xnext_ref[pl.ds(me * CHUNK, CHUNK), :] = yb
            for d in range(N_DEV):
                @pl.when(d != me)
                def _():
                    send_desc_ag(d).wait_send()
            for d in range(N_DEV):
                @pl.when(d != me)
                def _():
                    recv_desc_ag(d).wait_recv()
            return xnext_ref[:, :]

        def layer(xb, win_ref, wout_ref):
            w_in = win_ref[:, :].astype(jnp.bfloat16)
            h = jnp.dot(xb, w_in, preferred_element_type=jnp.float32)
            h = jnp.maximum(h, 0.0).astype(jnp.bfloat16)
            w_out = wout_ref[:, :].astype(jnp.bfloat16)
            return jnp.dot(h, w_out, preferred_element_type=jnp.float32)

        xb = x_ref[:, :].astype(jnp.bfloat16)
        weights = [(win0_ref, wout0_ref), (win1_ref, wout1_ref),
                   (win2_ref, wout2_ref)]
        for l, (win_ref, wout_ref) in enumerate(weights):
            p32 = layer(xb, win_ref, wout_ref)
            acc = reduce_scatter(p32)
            if l < 2:
                xb = all_gather(acc)
            else:
                out_ref[:, :] = acc

    return pl.pallas_call(
        body,
        out_shape=jax.ShapeDtypeStruct((CHUNK, D), jnp.float32),
        in_specs=[pl.BlockSpec(memory_space=pltpu.VMEM)] * 7,
        out_specs=pl.BlockSpec(memory_space=pltpu.VMEM),
        scratch_shapes=[
            pltpu.VMEM((B, D), jnp.bfloat16),
            pltpu.VMEM((N_DEV, CHUNK, D), jnp.bfloat16),
            pltpu.VMEM((CHUNK, D), jnp.bfloat16),
            pltpu.VMEM((B, D), jnp.bfloat16),
            pltpu.SemaphoreType.DMA((N_DEV,)),
            pltpu.SemaphoreType.DMA((N_DEV,)),
            pltpu.SemaphoreType.DMA((N_DEV,)),
        ],
        compiler_params=pltpu.CompilerParams(collective_id=0),
    )(x, Win0, Wout0, Win1, Wout1, Win2, Wout2)
